# baseline (speedup 1.0000x reference)
import functools

import jax
import jax.numpy as jnp
from jax import lax
from jax.experimental import pallas as pl
from jax.experimental.pallas import tpu as pltpu
from jax.experimental.pallas import tpu_sc as plsc

N = 10000
E = 320000
D = 128
BN_EPS = 1e-5

NC = 2
NS = 16
NW = NC * NS
EPW = E // NW
CHUNK = 80
NCHUNK = EPW // CHUNK
NSLOT = 4
NIDX = 5
N_PAD = 10240
ROWS_PER_TILE = N_PAD // NS
RB = 128
CNT_PER_TILE = N_PAD // NS


def _sc_agg_body(p_hbm, idx_hbm, out_acc, out_cnt,
                 ibuf, rows, cbuf, ones_v, acc_sh, cnt_sh,
                 gsem, isem, ssem, osem):
    cid = lax.axis_index("c")
    sid = lax.axis_index("s")
    wid = cid * NS + sid

    def zrow(k, _):
        r = k // (D // 16)
        c = (k % (D // 16)) * 16
        rows[r, pl.ds(c, 16)] = jnp.zeros((16,), jnp.float32)
        return 0
    lax.fori_loop(0, RB * (D // 16), zrow, 0)

    def zcnt(k, _):
        cbuf[pl.ds(k * 16, 16)] = jnp.zeros((16,), jnp.float32)
        return 0
    lax.fori_loop(0, CNT_PER_TILE // 16, zcnt, 0)

    def orow(k, _):
        ones_v[pl.ds(k * 16, 16)] = jnp.ones((16,), jnp.float32)
        return 0
    lax.fori_loop(0, 8, orow, 0)

    zb = rows.at[pl.ds(0, RB)]
    for k in range(ROWS_PER_TILE // RB):
        pltpu.sync_copy(zb, acc_sh.at[pl.ds(sid * ROWS_PER_TILE + k * RB, RB)])
    pltpu.sync_copy(cbuf, cnt_sh.at[pl.ds(sid * CNT_PER_TILE, CNT_PER_TILE)])
    plsc.subcore_barrier()

    def idx_slot(j):
        return ibuf.at[pl.ds((j % NIDX) * 2, 2)]

    def row_slot(j):
        return rows.at[pl.ds((j % NSLOT) * CHUNK, CHUNK)]

    def fire_idx(j):
        pltpu.async_copy(idx_hbm.at[wid, j], idx_slot(j), isem.at[j % NIDX])

    def wait_idx(j):
        pltpu.make_async_copy(
            idx_hbm.at[wid, j], idx_slot(j), isem.at[j % NIDX]).wait()

    def fire_gather(j):
        pltpu.async_copy(p_hbm.at[ibuf.at[(j % NIDX) * 2]], row_slot(j),
                         gsem.at[j % NSLOT])

    def wait_gather(j):
        pltpu.make_async_copy(p_hbm.at[ibuf.at[(j % NIDX) * 2]], row_slot(j),
                              gsem.at[j % NSLOT]).wait()

    def fire_scatter(j):
        dst = ibuf.at[(j % NIDX) * 2 + 1]
        pltpu.async_copy(row_slot(j), acc_sh.at[dst], ssem.at[j % NSLOT],
                         add=True)
        pltpu.async_copy(ones_v.at[pl.ds(0, CHUNK)], cnt_sh.at[dst],
                         osem.at[j % NSLOT], add=True)

    def wait_scatter(j):
        dst = ibuf.at[(j % NIDX) * 2 + 1]
        pltpu.make_async_copy(row_slot(j), acc_sh.at[dst],
                              ssem.at[j % NSLOT]).wait()
        pltpu.make_async_copy(ones_v.at[pl.ds(0, CHUNK)], cnt_sh.at[dst],
                              osem.at[j % NSLOT]).wait()

    fire_idx(0)
    fire_idx(1)
    fire_idx(2)
    wait_idx(0)
    fire_gather(0)
    wait_idx(1)
    fire_gather(1)
    for j in range(2):
        fire_idx(j + 3)
        wait_idx(j + 2)
        fire_gather(j + 2)
        wait_gather(j)
        fire_scatter(j)

    def chunk_body(j, _):
        wait_scatter(j - 2)
        fire_idx(j + 3)
        wait_idx(j + 2)
        fire_gather(j + 2)
        wait_gather(j)
        fire_scatter(j)
        return 0
    lax.fori_loop(2, NCHUNK - 3, chunk_body, 0)
    j = NCHUNK - 3
    wait_scatter(j - 2)
    wait_idx(j + 2)
    fire_gather(j + 2)
    wait_gather(j)
    fire_scatter(j)
    for j in range(NCHUNK - 2, NCHUNK):
        wait_scatter(j - 2)
        wait_gather(j)
        fire_scatter(j)
    wait_scatter(NCHUNK - 2)
    wait_scatter(NCHUNK - 1)
    plsc.subcore_barrier()

    for k in range(ROWS_PER_TILE // RB):
        r0 = sid * ROWS_PER_TILE + k * RB
        pltpu.sync_copy(acc_sh.at[pl.ds(r0, RB)], zb)
        pltpu.sync_copy(zb, out_acc.at[cid, pl.ds(r0, RB)])
    c0 = sid * CNT_PER_TILE
    pltpu.sync_copy(cnt_sh.at[pl.ds(c0, CNT_PER_TILE)], cbuf)
    pltpu.sync_copy(cbuf, out_cnt.at[cid, pl.ds(c0, CNT_PER_TILE)])


_sc_aggregate = functools.partial(
    pl.kernel,
    out_type=(
        jax.ShapeDtypeStruct((NC, N_PAD, D), jnp.float32),
        jax.ShapeDtypeStruct((NC, N_PAD), jnp.float32),
    ),
    mesh=plsc.VectorSubcoreMesh(
        core_axis_name="c", subcore_axis_name="s",
        num_cores=NC, num_subcores=NS,
    ),
    scratch_types=[
        pltpu.VMEM((NIDX * 2, CHUNK), jnp.int32),
        pltpu.VMEM((NSLOT * CHUNK, D), jnp.float32),
        pltpu.VMEM((CNT_PER_TILE,), jnp.float32),
        pltpu.VMEM((128,), jnp.float32),
        pltpu.VMEM_SHARED((N_PAD, D), jnp.float32),
        pltpu.VMEM_SHARED((N_PAD,), jnp.float32),
        pltpu.SemaphoreType.DMA((NSLOT,)),
        pltpu.SemaphoreType.DMA((NIDX,)),
        pltpu.SemaphoreType.DMA((NSLOT,)),
        pltpu.SemaphoreType.DMA((NSLOT,)),
    ],
)(_sc_agg_body)


BLK = 1024
GRID = (N + BLK - 1) // BLK


def _proj_body(x_ref, w_ref, a_ref, b_ref):
    xw = jnp.dot(x_ref[...], w_ref[...], preferred_element_type=jnp.float32)
    a_ref[...] = xw[:, :D]
    b_ref[...] = xw[:, D:]


def _project(x, wcat):
    return pl.pallas_call(
        _proj_body,
        grid=(GRID,),
        in_specs=[
            pl.BlockSpec((BLK, D), lambda i: (i, 0)),
            pl.BlockSpec((D, 2 * D), lambda i: (0, 0)),
        ],
        out_specs=[
            pl.BlockSpec((BLK, D), lambda i: (i, 0)),
            pl.BlockSpec((BLK, D), lambda i: (i, 0)),
        ],
        out_shape=[
            jax.ShapeDtypeStruct((N, D), jnp.float32),
            jax.ShapeDtypeStruct((N, D), jnp.float32),
        ],
    )(x, wcat)


def _mid_body(sp_ref, cnt_ref, xr_ref, b1_ref, bnm_ref, bna_ref, w2_ref,
              p2_ref, hr_ref):
    s = sp_ref[0] + sp_ref[1]
    cnt = cnt_ref[0] + cnt_ref[1]
    inv = 1.0 / jnp.maximum(cnt, 1.0)
    h = s * inv + b1_ref[...] + xr_ref[...]
    h = h * bnm_ref[...] + bna_ref[...]
    h = jnp.maximum(h, 0.0)
    hw = jnp.dot(h, w2_ref[...], preferred_element_type=jnp.float32)
    p2_ref[...] = hw[:, :D]
    hr_ref[...] = hw[:, D:]


def _mid(s1p, cnt2, xr, b1r, bnm, bna, w2cat):
    return pl.pallas_call(
        _mid_body,
        grid=(GRID,),
        in_specs=[
            pl.BlockSpec((2, BLK, D), lambda i: (0, i, 0)),
            pl.BlockSpec((2, BLK, 1), lambda i: (0, i, 0)),
            pl.BlockSpec((BLK, D), lambda i: (i, 0)),
            pl.BlockSpec((1, D), lambda i: (0, 0)),
            pl.BlockSpec((1, D), lambda i: (0, 0)),
            pl.BlockSpec((1, D), lambda i: (0, 0)),
            pl.BlockSpec((D, 2 * D), lambda i: (0, 0)),
        ],
        out_specs=[
            pl.BlockSpec((BLK, D), lambda i: (i, 0)),
            pl.BlockSpec((BLK, D), lambda i: (i, 0)),
        ],
        out_shape=[
            jax.ShapeDtypeStruct((N, D), jnp.float32),
            jax.ShapeDtypeStruct((N, D), jnp.float32),
        ],
    )(s1p, cnt2, xr, b1r, bnm, bna, w2cat)


def _final_body(sp_ref, cnt_ref, hr_ref, b2_ref, out_ref):
    s = sp_ref[0] + sp_ref[1]
    cnt = cnt_ref[0] + cnt_ref[1]
    inv = 1.0 / jnp.maximum(cnt, 1.0)
    out_ref[...] = s * inv + b2_ref[...] + hr_ref[...]


def _final(s2p, cnt2, hr, b2r):
    return pl.pallas_call(
        _final_body,
        grid=(GRID,),
        in_specs=[
            pl.BlockSpec((2, BLK, D), lambda i: (0, i, 0)),
            pl.BlockSpec((2, BLK, 1), lambda i: (0, i, 0)),
            pl.BlockSpec((BLK, D), lambda i: (i, 0)),
            pl.BlockSpec((1, D), lambda i: (0, 0)),
        ],
        out_specs=pl.BlockSpec((BLK, D), lambda i: (i, 0)),
        out_shape=jax.ShapeDtypeStruct((N, D), jnp.float32),
    )(s2p, cnt2, hr, b2r)


def kernel(x, edge_index, W1l, b1, W1r, gamma, beta, run_mean, run_var, W2l, b2, W2r):
    idx4 = jnp.stack(
        [edge_index[0].reshape(NW, NCHUNK, CHUNK),
         edge_index[1].reshape(NW, NCHUNK, CHUNK)], axis=2)

    bnm = (gamma * lax.rsqrt(run_var + BN_EPS)).reshape(1, D)
    bna = (beta - run_mean * bnm[0]).reshape(1, D)
    b1r = b1.reshape(1, D)
    b2r = b2.reshape(1, D)
    w1cat = jnp.concatenate([W1l, W1r], axis=1)
    w2cat = jnp.concatenate([W2l, W2r], axis=1)

    p1, xr = _project(x, w1cat)
    s1p, cntp = _sc_aggregate(p1, idx4)
    cnt2 = cntp[:, :, None]

    p2, hr = _mid(s1p, cnt2, xr, b1r, bnm, bna, w2cat)

    s2p, _ = _sc_aggregate(p2, idx4)
    return _final(s2p, cnt2, hr, b2r)


# speedup vs baseline: 1.0891x; 1.0891x over previous
import functools

import jax
import jax.numpy as jnp
from jax import lax
from jax.experimental import pallas as pl
from jax.experimental.pallas import tpu as pltpu
from jax.experimental.pallas import tpu_sc as plsc

N = 10000
E = 320000
D = 128
BN_EPS = 1e-5

NC = 2
NS = 16
NW = NC * NS
EPW = E // NW
CHUNK = 80
NCHUNK = EPW // CHUNK
NSLOT = 3
NIDX = 4
N_PAD = 10240
ROWS_PER_TILE = N_PAD // NS
RB = 128
CNT_PER_TILE = N_PAD // NS


def _sc_agg_body(p_hbm, eflat_hbm, out_acc, out_cnt,
                 sbuf, dbuf, rows, cbuf, ones_v, acc_sh, cnt_sh,
                 gsem, isem, ssem, osem):
    cid = lax.axis_index("c")
    sid = lax.axis_index("s")
    wid = cid * NS + sid

    def zrow(k, _):
        r = k // (D // 16)
        c = (k % (D // 16)) * 16
        rows[r, pl.ds(c, 16)] = jnp.zeros((16,), jnp.float32)
        return 0
    lax.fori_loop(0, RB * (D // 16), zrow, 0)

    def zcnt(k, _):
        cbuf[pl.ds(k * 16, 16)] = jnp.zeros((16,), jnp.float32)
        return 0
    lax.fori_loop(0, CNT_PER_TILE // 16, zcnt, 0)

    def orow(k, _):
        ones_v[pl.ds(k * 16, 16)] = jnp.ones((16,), jnp.float32)
        return 0
    lax.fori_loop(0, 8, orow, 0)

    zb = rows.at[pl.ds(0, RB)]
    for k in range(ROWS_PER_TILE // RB):
        pltpu.sync_copy(zb, acc_sh.at[pl.ds(sid * ROWS_PER_TILE + k * RB, RB)])
    pltpu.sync_copy(cbuf, cnt_sh.at[pl.ds(sid * CNT_PER_TILE, CNT_PER_TILE)])
    plsc.subcore_barrier()

    def row_slot(j):
        return rows.at[pl.ds((j % NSLOT) * CHUNK, CHUNK)]

    def fire_idx(j):
        off = pl.multiple_of(wid * EPW + j * CHUNK, 8)
        pltpu.async_copy(eflat_hbm.at[pl.ds(off, CHUNK)],
                         sbuf.at[j % NIDX], isem.at[j % NIDX])
        off2 = pl.multiple_of(E + wid * EPW + j * CHUNK, 8)
        pltpu.async_copy(eflat_hbm.at[pl.ds(off2, CHUNK)],
                         dbuf.at[j % NIDX], isem.at[j % NIDX])

    def wait_idx(j):
        off = pl.multiple_of(wid * EPW + j * CHUNK, 8)
        pltpu.make_async_copy(eflat_hbm.at[pl.ds(off, CHUNK)],
                              sbuf.at[j % NIDX], isem.at[j % NIDX]).wait()
        pltpu.make_async_copy(eflat_hbm.at[pl.ds(off, CHUNK)],
                              dbuf.at[j % NIDX], isem.at[j % NIDX]).wait()

    def fire_gather(j):
        pltpu.async_copy(p_hbm.at[sbuf.at[j % NIDX]], row_slot(j),
                         gsem.at[j % NSLOT])

    def wait_gather(j):
        pltpu.make_async_copy(p_hbm.at[sbuf.at[j % NIDX]], row_slot(j),
                              gsem.at[j % NSLOT]).wait()

    def fire_scatter(j):
        dst = dbuf.at[j % NIDX]
        pltpu.async_copy(row_slot(j), acc_sh.at[dst], ssem.at[j % NSLOT],
                         add=True)
        pltpu.async_copy(ones_v.at[pl.ds(0, CHUNK)], cnt_sh.at[dst],
                         osem.at[j % NSLOT], add=True)

    def wait_scatter(j):
        dst = dbuf.at[j % NIDX]
        pltpu.make_async_copy(row_slot(j), acc_sh.at[dst],
                              ssem.at[j % NSLOT]).wait()
        pltpu.make_async_copy(ones_v.at[pl.ds(0, CHUNK)], cnt_sh.at[dst],
                              osem.at[j % NSLOT]).wait()

    fire_idx(0)
    fire_idx(1)
    wait_idx(0)
    fire_gather(0)
    for j in range(2):
        fire_idx(j + 2)
        wait_idx(j + 1)
        fire_gather(j + 1)
        wait_gather(j)
        fire_scatter(j)

    def chunk_body(j, _):
        wait_scatter(j - 2)
        fire_idx(j + 2)
        wait_idx(j + 1)
        fire_gather(j + 1)
        wait_gather(j)
        fire_scatter(j)
        return 0
    lax.fori_loop(2, NCHUNK - 2, chunk_body, 0)
    for j in range(NCHUNK - 2, NCHUNK):
        wait_scatter(j - 2)
        if j + 1 < NCHUNK:
            wait_idx(j + 1)
            fire_gather(j + 1)
        wait_gather(j)
        fire_scatter(j)
    wait_scatter(NCHUNK - 2)
    wait_scatter(NCHUNK - 1)
    plsc.subcore_barrier()

    for k in range(ROWS_PER_TILE // RB):
        r0 = sid * ROWS_PER_TILE + k * RB
        pltpu.sync_copy(acc_sh.at[pl.ds(r0, RB)], zb)
        pltpu.sync_copy(zb, out_acc.at[cid, pl.ds(r0, RB)])
    c0 = sid * CNT_PER_TILE
    pltpu.sync_copy(cnt_sh.at[pl.ds(c0, CNT_PER_TILE)], cbuf)
    pltpu.sync_copy(cbuf, out_cnt.at[cid, pl.ds(c0, CNT_PER_TILE)])


_sc_aggregate = functools.partial(
    pl.kernel,
    out_type=(
        jax.ShapeDtypeStruct((NC, N_PAD, D), jnp.float32),
        jax.ShapeDtypeStruct((NC, N_PAD), jnp.float32),
    ),
    mesh=plsc.VectorSubcoreMesh(
        core_axis_name="c", subcore_axis_name="s",
        num_cores=NC, num_subcores=NS,
    ),
    scratch_types=[
        pltpu.VMEM((NIDX, CHUNK), jnp.int32),
        pltpu.VMEM((NIDX, CHUNK), jnp.int32),
        pltpu.VMEM((NSLOT * CHUNK, D), jnp.float32),
        pltpu.VMEM((CNT_PER_TILE,), jnp.float32),
        pltpu.VMEM((128,), jnp.float32),
        pltpu.VMEM_SHARED((N_PAD, D), jnp.float32),
        pltpu.VMEM_SHARED((N_PAD,), jnp.float32),
        pltpu.SemaphoreType.DMA((NSLOT,)),
        pltpu.SemaphoreType.DMA((NIDX,)),
        pltpu.SemaphoreType.DMA((NSLOT,)),
        pltpu.SemaphoreType.DMA((NSLOT,)),
    ],
)(_sc_agg_body)


BLK = 1024
GRID = (N + BLK - 1) // BLK


def _proj_body(x_ref, w_ref, a_ref, b_ref):
    xw = jnp.dot(x_ref[...], w_ref[...], preferred_element_type=jnp.float32)
    a_ref[...] = xw[:, :D]
    b_ref[...] = xw[:, D:]


def _project(x, wcat):
    return pl.pallas_call(
        _proj_body,
        grid=(GRID,),
        in_specs=[
            pl.BlockSpec((BLK, D), lambda i: (i, 0)),
            pl.BlockSpec((D, 2 * D), lambda i: (0, 0)),
        ],
        out_specs=[
            pl.BlockSpec((BLK, D), lambda i: (i, 0)),
            pl.BlockSpec((BLK, D), lambda i: (i, 0)),
        ],
        out_shape=[
            jax.ShapeDtypeStruct((N, D), jnp.float32),
            jax.ShapeDtypeStruct((N, D), jnp.float32),
        ],
    )(x, wcat)


def _mid_body(sp_ref, cnt_ref, xr_ref, b1_ref, bnm_ref, bna_ref, w2_ref,
              p2_ref, hr_ref):
    s = sp_ref[0] + sp_ref[1]
    cnt = cnt_ref[0] + cnt_ref[1]
    inv = 1.0 / jnp.maximum(cnt, 1.0)
    h = s * inv + b1_ref[...] + xr_ref[...]
    h = h * bnm_ref[...] + bna_ref[...]
    h = jnp.maximum(h, 0.0)
    hw = jnp.dot(h, w2_ref[...], preferred_element_type=jnp.float32)
    p2_ref[...] = hw[:, :D]
    hr_ref[...] = hw[:, D:]


def _mid(s1p, cnt2, xr, b1r, bnm, bna, w2cat):
    return pl.pallas_call(
        _mid_body,
        grid=(GRID,),
        in_specs=[
            pl.BlockSpec((2, BLK, D), lambda i: (0, i, 0)),
            pl.BlockSpec((2, BLK, 1), lambda i: (0, i, 0)),
            pl.BlockSpec((BLK, D), lambda i: (i, 0)),
            pl.BlockSpec((1, D), lambda i: (0, 0)),
            pl.BlockSpec((1, D), lambda i: (0, 0)),
            pl.BlockSpec((1, D), lambda i: (0, 0)),
            pl.BlockSpec((D, 2 * D), lambda i: (0, 0)),
        ],
        out_specs=[
            pl.BlockSpec((BLK, D), lambda i: (i, 0)),
            pl.BlockSpec((BLK, D), lambda i: (i, 0)),
        ],
        out_shape=[
            jax.ShapeDtypeStruct((N, D), jnp.float32),
            jax.ShapeDtypeStruct((N, D), jnp.float32),
        ],
    )(s1p, cnt2, xr, b1r, bnm, bna, w2cat)


def _final_body(sp_ref, cnt_ref, hr_ref, b2_ref, out_ref):
    s = sp_ref[0] + sp_ref[1]
    cnt = cnt_ref[0] + cnt_ref[1]
    inv = 1.0 / jnp.maximum(cnt, 1.0)
    out_ref[...] = s * inv + b2_ref[...] + hr_ref[...]


def _final(s2p, cnt2, hr, b2r):
    return pl.pallas_call(
        _final_body,
        grid=(GRID,),
        in_specs=[
            pl.BlockSpec((2, BLK, D), lambda i: (0, i, 0)),
            pl.BlockSpec((2, BLK, 1), lambda i: (0, i, 0)),
            pl.BlockSpec((BLK, D), lambda i: (i, 0)),
            pl.BlockSpec((1, D), lambda i: (0, 0)),
        ],
        out_specs=pl.BlockSpec((BLK, D), lambda i: (i, 0)),
        out_shape=jax.ShapeDtypeStruct((N, D), jnp.float32),
    )(s2p, cnt2, hr, b2r)


def kernel(x, edge_index, W1l, b1, W1r, gamma, beta, run_mean, run_var, W2l, b2, W2r):
    eflat = edge_index.reshape(2 * E)

    bnm = (gamma * lax.rsqrt(run_var + BN_EPS)).reshape(1, D)
    bna = (beta - run_mean * bnm[0]).reshape(1, D)
    b1r = b1.reshape(1, D)
    b2r = b2.reshape(1, D)
    w1cat = jnp.concatenate([W1l, W1r], axis=1)
    w2cat = jnp.concatenate([W2l, W2r], axis=1)

    p1, xr = _project(x, w1cat)
    s1p, cntp = _sc_aggregate(p1, eflat)
    cnt2 = cntp[:, :, None]

    p2, hr = _mid(s1p, cnt2, xr, b1r, bnm, bna, w2cat)

    s2p, _ = _sc_aggregate(p2, eflat)
    return _final(s2p, cnt2, hr, b2r)


# speedup vs baseline: 1.1047x; 1.0143x over previous
import functools

import jax
import jax.numpy as jnp
from jax import lax
from jax.experimental import pallas as pl
from jax.experimental.pallas import tpu as pltpu
from jax.experimental.pallas import tpu_sc as plsc

N = 10000
E = 320000
D = 128
BN_EPS = 1e-5

NC = 2
NS = 16
NW = NC * NS
EPW = E // NW
CHUNK = 80
NCHUNK = EPW // CHUNK
NSLOT = 3
NIDX = 4
N_PAD = 10240
ROWS_PER_TILE = N_PAD // NS
RB = 128
CNT_PER_TILE = N_PAD // NS


def _sc_agg_body(p_hbm, eflat_hbm, out_acc, out_cnt,
                 sbuf, dbuf, rows, cbuf, ones_v, acc_sh, cnt_sh,
                 gsem, isem, ssem, osem):
    cid = lax.axis_index("c")
    sid = lax.axis_index("s")
    wid = cid * NS + sid

    def zrow(k, _):
        r = k // (D // 16)
        c = (k % (D // 16)) * 16
        rows[r, pl.ds(c, 16)] = jnp.zeros((16,), jnp.float32)
        return 0
    lax.fori_loop(0, RB * (D // 16), zrow, 0)

    def zcnt(k, _):
        cbuf[pl.ds(k * 16, 16)] = jnp.zeros((16,), jnp.float32)
        return 0
    lax.fori_loop(0, CNT_PER_TILE // 16, zcnt, 0)

    def orow(k, _):
        ones_v[pl.ds(k * 16, 16)] = jnp.ones((16,), jnp.float32)
        return 0
    lax.fori_loop(0, 8, orow, 0)

    zb = rows.at[pl.ds(0, RB)]
    for k in range(ROWS_PER_TILE // RB):
        pltpu.sync_copy(zb, acc_sh.at[pl.ds(sid * ROWS_PER_TILE + k * RB, RB)])
    pltpu.sync_copy(cbuf, cnt_sh.at[pl.ds(sid * CNT_PER_TILE, CNT_PER_TILE)])
    plsc.subcore_barrier()

    def row_slot(j):
        return rows.at[pl.ds((j % NSLOT) * CHUNK, CHUNK)]

    def fire_idx(j):
        off = pl.multiple_of(wid * EPW + j * CHUNK, 8)
        pltpu.async_copy(eflat_hbm.at[pl.ds(off, CHUNK)],
                         sbuf.at[j % NIDX], isem.at[j % NIDX])
        off2 = pl.multiple_of(E + wid * EPW + j * CHUNK, 8)
        pltpu.async_copy(eflat_hbm.at[pl.ds(off2, CHUNK)],
                         dbuf.at[j % NIDX], isem.at[j % NIDX])

    def wait_idx(j):
        off = pl.multiple_of(wid * EPW + j * CHUNK, 8)
        pltpu.make_async_copy(eflat_hbm.at[pl.ds(off, CHUNK)],
                              sbuf.at[j % NIDX], isem.at[j % NIDX]).wait()
        pltpu.make_async_copy(eflat_hbm.at[pl.ds(off, CHUNK)],
                              dbuf.at[j % NIDX], isem.at[j % NIDX]).wait()

    def fire_gather(j):
        pltpu.async_copy(p_hbm.at[sbuf.at[j % NIDX]], row_slot(j),
                         gsem.at[j % NSLOT])

    def wait_gather(j):
        pltpu.make_async_copy(p_hbm.at[sbuf.at[j % NIDX]], row_slot(j),
                              gsem.at[j % NSLOT]).wait()

    def fire_scatter(j):
        dst = dbuf.at[j % NIDX]
        pltpu.async_copy(row_slot(j), acc_sh.at[dst], ssem.at[j % NSLOT],
                         add=True)
        pltpu.async_copy(ones_v.at[pl.ds(0, CHUNK)], cnt_sh.at[dst],
                         osem.at[j % NSLOT], add=True)

    def wait_scatter(j):
        dst = dbuf.at[j % NIDX]
        pltpu.make_async_copy(row_slot(j), acc_sh.at[dst],
                              ssem.at[j % NSLOT]).wait()
        pltpu.make_async_copy(ones_v.at[pl.ds(0, CHUNK)], cnt_sh.at[dst],
                              osem.at[j % NSLOT]).wait()

    fire_idx(0)
    fire_idx(1)
    wait_idx(0)
    fire_gather(0)
    for j in range(2):
        fire_idx(j + 2)
        wait_idx(j + 1)
        fire_gather(j + 1)
        wait_gather(j)
        fire_scatter(j)

    def chunk_body(j, _):
        wait_scatter(j - 2)
        fire_idx(j + 2)
        wait_idx(j + 1)
        fire_gather(j + 1)
        wait_gather(j)
        fire_scatter(j)
        return 0
    lax.fori_loop(2, NCHUNK - 2, chunk_body, 0)
    for j in range(NCHUNK - 2, NCHUNK):
        wait_scatter(j - 2)
        if j + 1 < NCHUNK:
            wait_idx(j + 1)
            fire_gather(j + 1)
        wait_gather(j)
        fire_scatter(j)
    wait_scatter(NCHUNK - 2)
    wait_scatter(NCHUNK - 1)
    plsc.subcore_barrier()

    for k in range(ROWS_PER_TILE // RB):
        r0 = sid * ROWS_PER_TILE + k * RB
        pltpu.sync_copy(acc_sh.at[pl.ds(r0, RB)], zb)
        pltpu.sync_copy(zb, out_acc.at[cid, pl.ds(r0, RB)])
    c0 = sid * CNT_PER_TILE
    pltpu.sync_copy(cnt_sh.at[pl.ds(c0, CNT_PER_TILE)], cbuf)
    pltpu.sync_copy(cbuf, out_cnt.at[cid, pl.ds(c0, CNT_PER_TILE)])


_sc_aggregate = functools.partial(
    pl.kernel,
    out_type=(
        jax.ShapeDtypeStruct((NC, N_PAD, D), jnp.float32),
        jax.ShapeDtypeStruct((NC, N_PAD), jnp.float32),
    ),
    mesh=plsc.VectorSubcoreMesh(
        core_axis_name="c", subcore_axis_name="s",
        num_cores=NC, num_subcores=NS,
    ),
    scratch_types=[
        pltpu.VMEM((NIDX, CHUNK), jnp.int32),
        pltpu.VMEM((NIDX, CHUNK), jnp.int32),
        pltpu.VMEM((NSLOT * CHUNK, D), jnp.float32),
        pltpu.VMEM((CNT_PER_TILE,), jnp.float32),
        pltpu.VMEM((128,), jnp.float32),
        pltpu.VMEM_SHARED((N_PAD, D), jnp.float32),
        pltpu.VMEM_SHARED((N_PAD,), jnp.float32),
        pltpu.SemaphoreType.DMA((NSLOT,)),
        pltpu.SemaphoreType.DMA((NIDX,)),
        pltpu.SemaphoreType.DMA((NSLOT,)),
        pltpu.SemaphoreType.DMA((NSLOT,)),
    ],
)(_sc_agg_body)


BLK = 2048
GRID = (N + BLK - 1) // BLK


def _proj_body(x_ref, w_ref, a_ref, b_ref):
    xb = x_ref[...].astype(jnp.bfloat16)
    xw = jnp.dot(xb, w_ref[...], preferred_element_type=jnp.float32)
    a_ref[...] = xw[:, :D]
    b_ref[...] = xw[:, D:]


def _project(x, wcat):
    return pl.pallas_call(
        _proj_body,
        grid=(GRID,),
        in_specs=[
            pl.BlockSpec((BLK, D), lambda i: (i, 0)),
            pl.BlockSpec((D, 2 * D), lambda i: (0, 0)),
        ],
        out_specs=[
            pl.BlockSpec((BLK, D), lambda i: (i, 0)),
            pl.BlockSpec((BLK, D), lambda i: (i, 0)),
        ],
        out_shape=[
            jax.ShapeDtypeStruct((N, D), jnp.float32),
            jax.ShapeDtypeStruct((N, D), jnp.float32),
        ],
    )(x, wcat)


def _mid_body(sp_ref, cnt_ref, xr_ref, b1_ref, bnm_ref, bna_ref, w2_ref,
              p2_ref, hr_ref):
    s = sp_ref[0] + sp_ref[1]
    cnt = cnt_ref[0] + cnt_ref[1]
    inv = 1.0 / jnp.maximum(cnt, 1.0)
    h = s * inv + b1_ref[...] + xr_ref[...]
    h = h * bnm_ref[...] + bna_ref[...]
    h = jnp.maximum(h, 0.0)
    hw = jnp.dot(h.astype(jnp.bfloat16), w2_ref[...],
                 preferred_element_type=jnp.float32)
    p2_ref[...] = hw[:, :D]
    hr_ref[...] = hw[:, D:]


def _mid(s1p, cnt2, xr, b1r, bnm, bna, w2cat):
    return pl.pallas_call(
        _mid_body,
        grid=(GRID,),
        in_specs=[
            pl.BlockSpec((2, BLK, D), lambda i: (0, i, 0)),
            pl.BlockSpec((2, BLK, 1), lambda i: (0, i, 0)),
            pl.BlockSpec((BLK, D), lambda i: (i, 0)),
            pl.BlockSpec((1, D), lambda i: (0, 0)),
            pl.BlockSpec((1, D), lambda i: (0, 0)),
            pl.BlockSpec((1, D), lambda i: (0, 0)),
            pl.BlockSpec((D, 2 * D), lambda i: (0, 0)),
        ],
        out_specs=[
            pl.BlockSpec((BLK, D), lambda i: (i, 0)),
            pl.BlockSpec((BLK, D), lambda i: (i, 0)),
        ],
        out_shape=[
            jax.ShapeDtypeStruct((N, D), jnp.float32),
            jax.ShapeDtypeStruct((N, D), jnp.float32),
        ],
    )(s1p, cnt2, xr, b1r, bnm, bna, w2cat)


def _final_body(sp_ref, cnt_ref, hr_ref, b2_ref, out_ref):
    s = sp_ref[0] + sp_ref[1]
    cnt = cnt_ref[0] + cnt_ref[1]
    inv = 1.0 / jnp.maximum(cnt, 1.0)
    out_ref[...] = s * inv + b2_ref[...] + hr_ref[...]


def _final(s2p, cnt2, hr, b2r):
    return pl.pallas_call(
        _final_body,
        grid=(GRID,),
        in_specs=[
            pl.BlockSpec((2, BLK, D), lambda i: (0, i, 0)),
            pl.BlockSpec((2, BLK, 1), lambda i: (0, i, 0)),
            pl.BlockSpec((BLK, D), lambda i: (i, 0)),
            pl.BlockSpec((1, D), lambda i: (0, 0)),
        ],
        out_specs=pl.BlockSpec((BLK, D), lambda i: (i, 0)),
        out_shape=jax.ShapeDtypeStruct((N, D), jnp.float32),
    )(s2p, cnt2, hr, b2r)


def kernel(x, edge_index, W1l, b1, W1r, gamma, beta, run_mean, run_var, W2l, b2, W2r):
    eflat = edge_index.reshape(2 * E)

    bnm = (gamma * lax.rsqrt(run_var + BN_EPS)).reshape(1, D)
    bna = (beta - run_mean * bnm[0]).reshape(1, D)
    b1r = b1.reshape(1, D)
    b2r = b2.reshape(1, D)
    w1cat = jnp.concatenate([W1l, W1r], axis=1).astype(jnp.bfloat16)
    w2cat = jnp.concatenate([W2l, W2r], axis=1).astype(jnp.bfloat16)

    p1, xr = _project(x, w1cat)
    s1p, cntp = _sc_aggregate(p1, eflat)
    cnt2 = cntp[:, :, None]

    p2, hr = _mid(s1p, cnt2, xr, b1r, bnm, bna, w2cat)

    s2p, _ = _sc_aggregate(p2, eflat)
    return _final(s2p, cnt2, hr, b2r)


# speedup vs baseline: 1.1100x; 1.0048x over previous
import functools

import jax
import jax.numpy as jnp
from jax import lax
from jax.experimental import pallas as pl
from jax.experimental.pallas import tpu as pltpu
from jax.experimental.pallas import tpu_sc as plsc

N = 10000
E = 320000
D = 128
BN_EPS = 1e-5

NC = 2
NS = 16
NW = NC * NS
EPW = E // NW
CHUNK = 80
NCHUNK = EPW // CHUNK
NSLOT = 3
NIDX = 4
N_PAD = 10240
ROWS_PER_TILE = N_PAD // NS
RB = 128
CNT_PER_TILE = N_PAD // NS


def _sc_agg_body(p_hbm, eflat_hbm, out_acc, out_cnt,
                 sbuf, dbuf, rows, cbuf, ones_v, acc_sh, cnt_sh,
                 gsem, isem, ssem, osem):
    cid = lax.axis_index("c")
    sid = lax.axis_index("s")
    wid = cid * NS + sid

    for _j in range(2):
        _off = pl.multiple_of(wid * EPW + _j * CHUNK, 8)
        pltpu.async_copy(eflat_hbm.at[pl.ds(_off, CHUNK)],
                         sbuf.at[_j % NIDX], isem.at[_j % NIDX])
        _off2 = pl.multiple_of(E + wid * EPW + _j * CHUNK, 8)
        pltpu.async_copy(eflat_hbm.at[pl.ds(_off2, CHUNK)],
                         dbuf.at[_j % NIDX], isem.at[_j % NIDX])

    def zrow(k, _):
        r = k // (D // 16)
        c = (k % (D // 16)) * 16
        rows[r, pl.ds(c, 16)] = jnp.zeros((16,), jnp.float32)
        return 0
    lax.fori_loop(0, RB * (D // 16), zrow, 0)

    def zcnt(k, _):
        cbuf[pl.ds(k * 16, 16)] = jnp.zeros((16,), jnp.float32)
        return 0
    lax.fori_loop(0, CNT_PER_TILE // 16, zcnt, 0)

    def orow(k, _):
        ones_v[pl.ds(k * 16, 16)] = jnp.ones((16,), jnp.float32)
        return 0
    lax.fori_loop(0, 8, orow, 0)

    zb = rows.at[pl.ds(0, RB)]
    for k in range(ROWS_PER_TILE // RB):
        pltpu.sync_copy(zb, acc_sh.at[pl.ds(sid * ROWS_PER_TILE + k * RB, RB)])
    pltpu.sync_copy(cbuf, cnt_sh.at[pl.ds(sid * CNT_PER_TILE, CNT_PER_TILE)])
    plsc.subcore_barrier()

    def row_slot(j):
        return rows.at[pl.ds((j % NSLOT) * CHUNK, CHUNK)]

    def fire_idx(j):
        off = pl.multiple_of(wid * EPW + j * CHUNK, 8)
        pltpu.async_copy(eflat_hbm.at[pl.ds(off, CHUNK)],
                         sbuf.at[j % NIDX], isem.at[j % NIDX])
        off2 = pl.multiple_of(E + wid * EPW + j * CHUNK, 8)
        pltpu.async_copy(eflat_hbm.at[pl.ds(off2, CHUNK)],
                         dbuf.at[j % NIDX], isem.at[j % NIDX])

    def wait_idx(j):
        off = pl.multiple_of(wid * EPW + j * CHUNK, 8)
        pltpu.make_async_copy(eflat_hbm.at[pl.ds(off, CHUNK)],
                              sbuf.at[j % NIDX], isem.at[j % NIDX]).wait()
        pltpu.make_async_copy(eflat_hbm.at[pl.ds(off, CHUNK)],
                              dbuf.at[j % NIDX], isem.at[j % NIDX]).wait()

    def fire_gather(j):
        pltpu.async_copy(p_hbm.at[sbuf.at[j % NIDX]], row_slot(j),
                         gsem.at[j % NSLOT])

    def wait_gather(j):
        pltpu.make_async_copy(p_hbm.at[sbuf.at[j % NIDX]], row_slot(j),
                              gsem.at[j % NSLOT]).wait()

    def fire_scatter(j):
        dst = dbuf.at[j % NIDX]
        pltpu.async_copy(row_slot(j), acc_sh.at[dst], ssem.at[j % NSLOT],
                         add=True)
        pltpu.async_copy(ones_v.at[pl.ds(0, CHUNK)], cnt_sh.at[dst],
                         osem.at[j % NSLOT], add=True)

    def wait_scatter(j):
        dst = dbuf.at[j % NIDX]
        pltpu.make_async_copy(row_slot(j), acc_sh.at[dst],
                              ssem.at[j % NSLOT]).wait()
        pltpu.make_async_copy(ones_v.at[pl.ds(0, CHUNK)], cnt_sh.at[dst],
                              osem.at[j % NSLOT]).wait()

    wait_idx(0)
    fire_gather(0)
    for j in range(2):
        fire_idx(j + 2)
        wait_idx(j + 1)
        fire_gather(j + 1)
        wait_gather(j)
        fire_scatter(j)

    def chunk_body(j, _):
        wait_scatter(j - 2)
        fire_idx(j + 2)
        wait_idx(j + 1)
        fire_gather(j + 1)
        wait_gather(j)
        fire_scatter(j)
        return 0
    lax.fori_loop(2, NCHUNK - 2, chunk_body, 0)
    for j in range(NCHUNK - 2, NCHUNK):
        wait_scatter(j - 2)
        if j + 1 < NCHUNK:
            wait_idx(j + 1)
            fire_gather(j + 1)
        wait_gather(j)
        fire_scatter(j)
    wait_scatter(NCHUNK - 2)
    wait_scatter(NCHUNK - 1)
    plsc.subcore_barrier()

    for k in range(ROWS_PER_TILE // RB):
        r0 = sid * ROWS_PER_TILE + k * RB
        pltpu.sync_copy(acc_sh.at[pl.ds(r0, RB)], zb)
        pltpu.sync_copy(zb, out_acc.at[cid, pl.ds(r0, RB)])
    c0 = sid * CNT_PER_TILE
    pltpu.sync_copy(cnt_sh.at[pl.ds(c0, CNT_PER_TILE)], cbuf)
    pltpu.sync_copy(cbuf, out_cnt.at[cid, pl.ds(c0, CNT_PER_TILE)])


_sc_aggregate = functools.partial(
    pl.kernel,
    out_type=(
        jax.ShapeDtypeStruct((NC, N_PAD, D), jnp.float32),
        jax.ShapeDtypeStruct((NC, N_PAD), jnp.float32),
    ),
    mesh=plsc.VectorSubcoreMesh(
        core_axis_name="c", subcore_axis_name="s",
        num_cores=NC, num_subcores=NS,
    ),
    scratch_types=[
        pltpu.VMEM((NIDX, CHUNK), jnp.int32),
        pltpu.VMEM((NIDX, CHUNK), jnp.int32),
        pltpu.VMEM((NSLOT * CHUNK, D), jnp.float32),
        pltpu.VMEM((CNT_PER_TILE,), jnp.float32),
        pltpu.VMEM((128,), jnp.float32),
        pltpu.VMEM_SHARED((N_PAD, D), jnp.float32),
        pltpu.VMEM_SHARED((N_PAD,), jnp.float32),
        pltpu.SemaphoreType.DMA((NSLOT,)),
        pltpu.SemaphoreType.DMA((NIDX,)),
        pltpu.SemaphoreType.DMA((NSLOT,)),
        pltpu.SemaphoreType.DMA((NSLOT,)),
    ],
)(_sc_agg_body)


BLK = 2048
GRID = (N + BLK - 1) // BLK


def _proj_body(x_ref, w_ref, a_ref, b_ref):
    xb = x_ref[...].astype(jnp.bfloat16)
    xw = jnp.dot(xb, w_ref[...], preferred_element_type=jnp.float32)
    a_ref[...] = xw[:, :D]
    b_ref[...] = xw[:, D:]


def _project(x, wcat):
    return pl.pallas_call(
        _proj_body,
        grid=(GRID,),
        in_specs=[
            pl.BlockSpec((BLK, D), lambda i: (i, 0)),
            pl.BlockSpec((D, 2 * D), lambda i: (0, 0)),
        ],
        out_specs=[
            pl.BlockSpec((BLK, D), lambda i: (i, 0)),
            pl.BlockSpec((BLK, D), lambda i: (i, 0)),
        ],
        out_shape=[
            jax.ShapeDtypeStruct((N, D), jnp.float32),
            jax.ShapeDtypeStruct((N, D), jnp.float32),
        ],
    )(x, wcat)


def _mid_body(sp_ref, cnt_ref, xr_ref, b1_ref, bnm_ref, bna_ref, w2_ref,
              p2_ref, hr_ref):
    s = sp_ref[0] + sp_ref[1]
    cnt = cnt_ref[0] + cnt_ref[1]
    inv = 1.0 / jnp.maximum(cnt, 1.0)
    h = s * inv + b1_ref[...] + xr_ref[...]
    h = h * bnm_ref[...] + bna_ref[...]
    h = jnp.maximum(h, 0.0)
    hw = jnp.dot(h.astype(jnp.bfloat16), w2_ref[...],
                 preferred_element_type=jnp.float32)
    p2_ref[...] = hw[:, :D]
    hr_ref[...] = hw[:, D:]


def _mid(s1p, cnt2, xr, b1r, bnm, bna, w2cat):
    return pl.pallas_call(
        _mid_body,
        grid=(GRID,),
        in_specs=[
            pl.BlockSpec((2, BLK, D), lambda i: (0, i, 0)),
            pl.BlockSpec((2, BLK, 1), lambda i: (0, i, 0)),
            pl.BlockSpec((BLK, D), lambda i: (i, 0)),
            pl.BlockSpec((1, D), lambda i: (0, 0)),
            pl.BlockSpec((1, D), lambda i: (0, 0)),
            pl.BlockSpec((1, D), lambda i: (0, 0)),
            pl.BlockSpec((D, 2 * D), lambda i: (0, 0)),
        ],
        out_specs=[
            pl.BlockSpec((BLK, D), lambda i: (i, 0)),
            pl.BlockSpec((BLK, D), lambda i: (i, 0)),
        ],
        out_shape=[
            jax.ShapeDtypeStruct((N, D), jnp.float32),
            jax.ShapeDtypeStruct((N, D), jnp.float32),
        ],
    )(s1p, cnt2, xr, b1r, bnm, bna, w2cat)


def _final_body(sp_ref, cnt_ref, hr_ref, b2_ref, out_ref):
    s = sp_ref[0] + sp_ref[1]
    cnt = cnt_ref[0] + cnt_ref[1]
    inv = 1.0 / jnp.maximum(cnt, 1.0)
    out_ref[...] = s * inv + b2_ref[...] + hr_ref[...]


def _final(s2p, cnt2, hr, b2r):
    return pl.pallas_call(
        _final_body,
        grid=(GRID,),
        in_specs=[
            pl.BlockSpec((2, BLK, D), lambda i: (0, i, 0)),
            pl.BlockSpec((2, BLK, 1), lambda i: (0, i, 0)),
            pl.BlockSpec((BLK, D), lambda i: (i, 0)),
            pl.BlockSpec((1, D), lambda i: (0, 0)),
        ],
        out_specs=pl.BlockSpec((BLK, D), lambda i: (i, 0)),
        out_shape=jax.ShapeDtypeStruct((N, D), jnp.float32),
    )(s2p, cnt2, hr, b2r)


def kernel(x, edge_index, W1l, b1, W1r, gamma, beta, run_mean, run_var, W2l, b2, W2r):
    eflat = edge_index.reshape(2 * E)

    bnm = (gamma * lax.rsqrt(run_var + BN_EPS)).reshape(1, D)
    bna = (beta - run_mean * bnm[0]).reshape(1, D)
    b1r = b1.reshape(1, D)
    b2r = b2.reshape(1, D)
    w1cat = jnp.concatenate([W1l, W1r], axis=1).astype(jnp.bfloat16)
    w2cat = jnp.concatenate([W2l, W2r], axis=1).astype(jnp.bfloat16)

    p1, xr = _project(x, w1cat)
    s1p, cntp = _sc_aggregate(p1, eflat)
    cnt2 = cntp[:, :, None]

    p2, hr = _mid(s1p, cnt2, xr, b1r, bnm, bna, w2cat)

    s2p, _ = _sc_aggregate(p2, eflat)
    return _final(s2p, cnt2, hr, b2r)
